# Initial kernel scaffold; baseline (speedup 1.0000x reference)
import jax
import jax.numpy as jnp
from jax.experimental import pallas as pl


def kernel(x, edge_index, edge_attr, batch, le1_W, le1_b, n1a_W, n1a_b, n1b_W, n1b_b, le2_W, le2_b, n2a_W, n2a_b, n2b_W, n2b_b, ffn_W, ffn_b):
    raise NotImplementedError("write your pallas kernel here")


# speedup vs baseline: 1.9430x; 1.9430x over previous
import functools

import jax
import jax.numpy as jnp
from jax import lax
from jax.experimental import pallas as pl
from jax.experimental.pallas import tpu as pltpu
from jax.experimental.pallas import tpu_sc as plsc

N = 10000
E = 320000
D = 128
ED = 16
G = 16
C = 10

NC = 2
NS = 16
NW = NC * NS
CH = 128
NCH = 80
IG = 8
E_PAD = NW * NCH * CH
N_PAD = 10112
ZB = 8
ZCOPIES = (N_PAD // NS) // ZB
OUT_ROWS = N_PAD // NS

RA = 2048
RN = 1000


def _edge_embed_body(a_ref, w1_ref, b1_ref, w2_ref, b2_ref, e1_ref, e2_ref):
    a = a_ref[...]
    e1_ref[...] = jnp.dot(a, w1_ref[...], preferred_element_type=jnp.float32) + b1_ref[...]
    e2_ref[...] = jnp.dot(a, w2_ref[...], preferred_element_type=jnp.float32) + b2_ref[...]


def _edge_embed(attr_p, w1t, b1, w2t, b2):
    grid = (E_PAD // RA,)
    return pl.pallas_call(
        _edge_embed_body,
        grid=grid,
        in_specs=[
            pl.BlockSpec((RA, ED), lambda i: (i, 0)),
            pl.BlockSpec((ED, D), lambda i: (0, 0)),
            pl.BlockSpec((1, D), lambda i: (0, 0)),
            pl.BlockSpec((ED, D), lambda i: (0, 0)),
            pl.BlockSpec((1, D), lambda i: (0, 0)),
        ],
        out_specs=[
            pl.BlockSpec((RA, D), lambda i: (i, 0)),
            pl.BlockSpec((RA, D), lambda i: (i, 0)),
        ],
        out_shape=[
            jax.ShapeDtypeStruct((E_PAD, D), jnp.float32),
            jax.ShapeDtypeStruct((E_PAD, D), jnp.float32),
        ],
    )(attr_p, w1t, b1.reshape(1, D), w2t, b2.reshape(1, D))


def _sc_gather_scatter(table, src2d, dst2d, e):
    mesh = plsc.VectorSubcoreMesh(core_axis_name="c", subcore_axis_name="s")

    @functools.partial(
        pl.kernel,
        out_type=jax.ShapeDtypeStruct((NC, N_PAD, D), jnp.float32),
        mesh=mesh,
        scratch_types=[
            pltpu.VMEM((IG, CH), jnp.int32),
            pltpu.VMEM((IG, CH), jnp.int32),
            pltpu.VMEM((CH, D), jnp.float32),
            pltpu.VMEM((CH, D), jnp.float32),
            pltpu.VMEM((ZB, D), jnp.float32),
            pltpu.VMEM_SHARED((N_PAD, D), jnp.float32),
            pltpu.SemaphoreType.DMA,
        ],
    )
    def k(table_hbm, src_hbm, dst_hbm, e_hbm, out_hbm,
          src_v, dst_v, e_v, xg_v, zb_v, agg_sh, sem):
        c = lax.axis_index("c")
        s = lax.axis_index("s")
        wid = s * NC + c

        def _zrow(i, carry):
            for cc in range(D // 16):
                zb_v[i, pl.ds(cc * 16, 16)] = jnp.zeros((16,), jnp.float32)
            return carry
        lax.fori_loop(0, ZB, _zrow, 0)

        def _zcp(t, carry):
            pltpu.sync_copy(zb_v, agg_sh.at[pl.ds(s * (N_PAD // NS) + t * ZB, ZB)])
            return carry
        lax.fori_loop(0, ZCOPIES, _zcp, 0)
        plsc.subcore_barrier()

        ebase = wid * (NCH * CH)

        def _group(g, carry):
            pltpu.sync_copy(src_hbm.at[pl.ds(wid * NCH + g * IG, IG)], src_v)
            pltpu.sync_copy(dst_hbm.at[pl.ds(wid * NCH + g * IG, IG)], dst_v)

            def _chunk(j, carry2):
                gcp = pltpu.async_copy(table_hbm.at[src_v.at[j]], xg_v, sem)
                pltpu.sync_copy(e_hbm.at[pl.ds(ebase + (g * IG + j) * CH, CH)], e_v)
                gcp.wait()

                def _row(r, rc):
                    for cc in range(D // 16):
                        sl = pl.ds(cc * 16, 16)
                        e_v[r, sl] = jnp.maximum(e_v[r, sl] + xg_v[r, sl], 0.0)
                    return rc
                lax.fori_loop(0, CH, _row, 0)
                pltpu.sync_copy(e_v, agg_sh.at[dst_v.at[j]], add=True)
                return carry2
            lax.fori_loop(0, IG, _chunk, 0)
            return carry
        lax.fori_loop(0, NCH // IG, _group, 0)

        plsc.subcore_barrier()
        pltpu.sync_copy(agg_sh.at[pl.ds(s * OUT_ROWS, OUT_ROWS)],
                        out_hbm.at[c].at[pl.ds(s * OUT_ROWS, OUT_ROWS)])

    return k(table, src2d, dst2d, e)


def _node_mlp_body(x_ref, a0_ref, a1_ref, wa_ref, ba_ref, wb_ref, bb_ref, o_ref):
    h = x_ref[...] + a0_ref[0] + a1_ref[0]
    h = jnp.dot(h, wa_ref[...], preferred_element_type=jnp.float32) + ba_ref[...]
    h = jnp.dot(h, wb_ref[...], preferred_element_type=jnp.float32) + bb_ref[...]
    o_ref[...] = jnp.maximum(h, 0.0)


def _node_mlp(x, agg, wat, ba, wbt, bb):
    grid = (N // RN,)
    return pl.pallas_call(
        _node_mlp_body,
        grid=grid,
        in_specs=[
            pl.BlockSpec((RN, D), lambda i: (i, 0)),
            pl.BlockSpec((1, RN, D), lambda i: (0, i, 0)),
            pl.BlockSpec((1, RN, D), lambda i: (1, i, 0)),
            pl.BlockSpec((D, D), lambda i: (0, 0)),
            pl.BlockSpec((1, D), lambda i: (0, 0)),
            pl.BlockSpec((D, D), lambda i: (0, 0)),
            pl.BlockSpec((1, D), lambda i: (0, 0)),
        ],
        out_specs=pl.BlockSpec((RN, D), lambda i: (i, 0)),
        out_shape=jax.ShapeDtypeStruct((N, D), jnp.float32),
    )(x, agg, agg, wat, ba.reshape(1, D), wbt, bb.reshape(1, D))


def _node_mlp_pool_body(x_ref, a0_ref, a1_ref, wa_ref, ba_ref, wb_ref, bb_ref,
                        batch_ref, sums_ref, cnts_ref):
    i = pl.program_id(0)
    h = x_ref[...] + a0_ref[0] + a1_ref[0]
    h = jnp.dot(h, wa_ref[...], preferred_element_type=jnp.float32) + ba_ref[...]
    h = jnp.dot(h, wb_ref[...], preferred_element_type=jnp.float32) + bb_ref[...]
    h = jnp.maximum(h, 0.0)
    b = batch_ref[0, 0, :]
    onehot = (b[:, None] == lax.broadcasted_iota(jnp.int32, (RN, G), 1)).astype(jnp.float32)
    part = lax.dot_general(onehot, h, (((0,), (0,)), ((), ())),
                           preferred_element_type=jnp.float32)
    cnt = jnp.broadcast_to(jnp.sum(onehot, axis=0)[:, None], (G, D))

    @pl.when(i == 0)
    def _():
        sums_ref[...] = jnp.zeros_like(sums_ref)
        cnts_ref[...] = jnp.zeros_like(cnts_ref)
    sums_ref[...] += part
    cnts_ref[...] += cnt


def _node_mlp_pool(h1, agg, wat, ba, wbt, bb, batch3d):
    grid = (N // RN,)
    return pl.pallas_call(
        _node_mlp_pool_body,
        grid=grid,
        in_specs=[
            pl.BlockSpec((RN, D), lambda i: (i, 0)),
            pl.BlockSpec((1, RN, D), lambda i: (0, i, 0)),
            pl.BlockSpec((1, RN, D), lambda i: (1, i, 0)),
            pl.BlockSpec((D, D), lambda i: (0, 0)),
            pl.BlockSpec((1, D), lambda i: (0, 0)),
            pl.BlockSpec((D, D), lambda i: (0, 0)),
            pl.BlockSpec((1, D), lambda i: (0, 0)),
            pl.BlockSpec((1, 1, RN), lambda i: (i, 0, 0)),
        ],
        out_specs=[
            pl.BlockSpec((G, D), lambda i: (0, 0)),
            pl.BlockSpec((G, D), lambda i: (0, 0)),
        ],
        out_shape=[
            jax.ShapeDtypeStruct((G, D), jnp.float32),
            jax.ShapeDtypeStruct((G, D), jnp.float32),
        ],
    )(h1, agg, agg, wat, ba.reshape(1, D), wbt, bb.reshape(1, D), batch3d)


def _head_body(sums_ref, cnts_ref, w_ref, b_ref, o_ref):
    pooled = sums_ref[...] / jnp.maximum(cnts_ref[...], 1.0)
    logits = lax.dot_general(pooled, w_ref[...], (((1,), (1,)), ((), ())),
                             preferred_element_type=jnp.float32) + b_ref[...]
    mask = lax.broadcasted_iota(jnp.int32, (G, G), 1) < C
    logits = jnp.where(mask, logits, -1e30)
    m = jnp.max(logits, axis=1, keepdims=True)
    ez = jnp.exp(logits - m)
    o_ref[...] = ez / jnp.sum(ez, axis=1, keepdims=True)


def _head(sums, cnts, ffn_Wp, ffn_bp):
    return pl.pallas_call(
        _head_body,
        in_specs=[
            pl.BlockSpec((G, D), lambda: (0, 0)),
            pl.BlockSpec((G, D), lambda: (0, 0)),
            pl.BlockSpec((G, D), lambda: (0, 0)),
            pl.BlockSpec((1, G), lambda: (0, 0)),
        ],
        out_specs=pl.BlockSpec((G, G), lambda: (0, 0)),
        out_shape=jax.ShapeDtypeStruct((G, G), jnp.float32),
    )(sums, cnts, ffn_Wp, ffn_bp)


def kernel(x, edge_index, edge_attr, batch,
           le1_W, le1_b, n1a_W, n1a_b, n1b_W, n1b_b,
           le2_W, le2_b, n2a_W, n2a_b, n2b_W, n2b_b,
           ffn_W, ffn_b):
    pad = E_PAD - E
    src2d = jnp.concatenate([edge_index[0], jnp.zeros((pad,), jnp.int32)]).reshape(NW * NCH, CH)
    dst2d = jnp.concatenate([edge_index[1], jnp.full((pad,), N, jnp.int32)]).reshape(NW * NCH, CH)
    attr_p = jnp.concatenate([edge_attr, jnp.zeros((pad, ED), jnp.float32)])
    batch3d = batch.reshape(N // RN, 1, RN)

    e1, e2 = _edge_embed(attr_p, le1_W.T, le1_b, le2_W.T, le2_b)

    agg1 = _sc_gather_scatter(x, src2d, dst2d, e1)
    h1 = _node_mlp(x, agg1, n1a_W.T, n1a_b, n1b_W.T, n1b_b)

    agg2 = _sc_gather_scatter(h1, src2d, dst2d, e2)
    ffn_Wp = jnp.concatenate([ffn_W, jnp.zeros((G - C, D), jnp.float32)])
    ffn_bp = jnp.concatenate([ffn_b, jnp.zeros((G - C,), jnp.float32)]).reshape(1, G)
    sums, cnts = _node_mlp_pool(h1, agg2, n2a_W.T, n2a_b, n2b_W.T, n2b_b, batch3d)

    out = _head(sums, cnts, ffn_Wp, ffn_bp)
    return out[:, :C]
